# baseline (speedup 1.0000x reference)
import jax
import jax.numpy as jnp
from jax import lax
from jax.experimental import pallas as pl
from jax.experimental.pallas import tpu as pltpu
from jax.experimental.pallas import tpu_sc as plsc

N, E, D, R = 10000, 320000, 128, 8

NC, NS, L = 2, 16, 16
NW = NC * NS
CH = 128
GRP = 8
CPW = 80
GPW = CPW // GRP
EP = NW * CPW * CH
NGRP = EP // (GRP * CH)
NP = 10240
ROWS_PER_TILE = NP // NS

BN = 2000


def _xw_body(x_ref, w_ref, y_ref):
    y_ref[0] = jnp.dot(x_ref[...], w_ref[0], preferred_element_type=jnp.float32)


def _transform_table(x, basis):
    return pl.pallas_call(
        _xw_body,
        grid=(R, N // BN),
        in_specs=[
            pl.BlockSpec((BN, D), lambda r, nb: (nb, 0)),
            pl.BlockSpec((1, D, D), lambda r, nb: (r, 0, 0)),
        ],
        out_specs=pl.BlockSpec((1, BN, D), lambda r, nb: (r, nb, 0)),
        out_shape=jax.ShapeDtypeStruct((R, N, D), jnp.float32),
    )(x, basis)


def _sc_body(table, ridx3d, dst3d, slabs,
             acc, idxr, dstr, rows0, rows1, sem0, sem1):
    c = lax.axis_index("c")
    s = lax.axis_index("s")
    w = c * NS + s
    g0 = w * GPW

    def _zrow(i, carry):
        for j in range(D // L):
            rows0[i, pl.ds(j * L, L)] = jnp.zeros((L,), jnp.float32)
        return carry
    lax.fori_loop(0, CH, _zrow, 0)
    for t in range(ROWS_PER_TILE // CH):
        pltpu.sync_copy(rows0, acc.at[pl.ds(s * ROWS_PER_TILE + t * CH, CH)])

    pltpu.sync_copy(ridx3d.at[g0], idxr.at[0])
    pltpu.sync_copy(dst3d.at[g0], dstr.at[0])

    plsc.subcore_barrier()

    rowbufs = (rows0, rows1)
    sems = (sem0, sem1)

    def _gather(p, j, b):
        pltpu.async_copy(table.at[idxr.at[p, j]], rowbufs[b], sems[b])

    def _drain(p, j, b):
        pltpu.make_async_copy(table.at[idxr.at[p, j]],
                              rowbufs[b], sems[b]).wait()
        pltpu.sync_copy(rowbufs[b], acc.at[dstr.at[p, j]], add=True)

    def _group(g, p, stage_next, last):
        if stage_next:
            pltpu.sync_copy(ridx3d.at[g + 1], idxr.at[1 - p])
            pltpu.sync_copy(dst3d.at[g + 1], dstr.at[1 - p])
        for j in range(GRP):
            if not (last and j == GRP - 1):
                if j == GRP - 1:
                    _gather(1 - p, 0, (j + 1) % 2)
                else:
                    _gather(p, j + 1, (j + 1) % 2)
            _drain(p, j, j % 2)

    _gather(0, 0, 0)

    def _pair(m, carry):
        g = g0 + 2 * m
        _group(g, 0, True, False)
        _group(g + 1, 1, True, False)
        return carry
    lax.fori_loop(0, GPW // 2 - 1, _pair, 0)

    _group(g0 + GPW - 2, 0, True, False)
    _group(g0 + GPW - 1, 1, False, True)

    plsc.subcore_barrier()

    pltpu.sync_copy(acc.at[pl.ds(s * ROWS_PER_TILE, ROWS_PER_TILE)],
                    slabs.at[c, pl.ds(s * ROWS_PER_TILE, ROWS_PER_TILE)])


def _aggregate(table, ridx3d, dst3d):
    fn = pl.kernel(
        _sc_body,
        out_type=jax.ShapeDtypeStruct((NC, NP, D), jnp.float32),
        mesh=plsc.VectorSubcoreMesh(core_axis_name="c", subcore_axis_name="s"),
        scratch_types=[
            pltpu.VMEM_SHARED((NP, D), jnp.float32),
            pltpu.VMEM((2, GRP, CH), jnp.int32),
            pltpu.VMEM((2, GRP, CH), jnp.int32),
            pltpu.VMEM((CH, D), jnp.float32),
            pltpu.VMEM((CH, D), jnp.float32),
            pltpu.SemaphoreType.DMA,
            pltpu.SemaphoreType.DMA,
        ],
    )
    return fn(table, ridx3d, dst3d)


def _fin_body(s_ref, x_ref, root_ref, bias_ref, out_ref):
    out_ref[...] = (s_ref[0] + s_ref[1] + bias_ref[...]
                    + jnp.dot(x_ref[...], root_ref[...],
                              preferred_element_type=jnp.float32))


def _finalize(slabs, x, root, bias2d):
    return pl.pallas_call(
        _fin_body,
        grid=(N // BN,),
        in_specs=[
            pl.BlockSpec((NC, BN, D), lambda nb: (0, nb, 0)),
            pl.BlockSpec((BN, D), lambda nb: (nb, 0)),
            pl.BlockSpec((D, D), lambda nb: (0, 0)),
            pl.BlockSpec((1, D), lambda nb: (0, 0)),
        ],
        out_specs=pl.BlockSpec((BN, D), lambda nb: (nb, 0)),
        out_shape=jax.ShapeDtypeStruct((N, D), jnp.float32),
    )(slabs, x, root, bias2d)


def kernel(x, edge_index, edge_type, basis, root, bias):
    src = edge_index[0]
    dst = edge_index[1]
    ridx = edge_type * N + src
    pad = EP - E
    ridx3d = jnp.concatenate(
        [ridx, jnp.zeros((pad,), jnp.int32)]).reshape(NGRP, GRP, CH)
    pad_dst = N + jnp.arange(pad, dtype=jnp.int32) % (NP - N)
    dst3d = jnp.concatenate([dst, pad_dst]).reshape(NGRP, GRP, CH)

    table = _transform_table(x, basis).reshape(R * N, D)
    slabs = _aggregate(table, ridx3d, dst3d)
    return _finalize(slabs, x, root, bias.reshape(1, D))


# speedup vs baseline: 2.8131x; 2.8131x over previous
import jax
import jax.numpy as jnp
from jax import lax
from jax.experimental import pallas as pl
from jax.experimental.pallas import tpu as pltpu
from jax.experimental.pallas import tpu_sc as plsc

N, E, D, R = 10000, 320000, 128, 8

NC, NS, L = 2, 16, 16
NW = NC * NS
CH = 128
GRP = 8
CPW = 80
GPW = CPW // GRP
EP = NW * CPW * CH
NGRP = EP // (GRP * CH)
NP = 10240
ROWS_PER_TILE = NP // NS

BN = 2000


def _xw_body(x_ref, w_ref, y_ref):
    y_ref[0] = jnp.dot(x_ref[...], w_ref[0], preferred_element_type=jnp.float32)


def _transform_table(x, basis):
    return pl.pallas_call(
        _xw_body,
        grid=(R, N // BN),
        in_specs=[
            pl.BlockSpec((BN, D), lambda r, nb: (nb, 0)),
            pl.BlockSpec((1, D, D), lambda r, nb: (r, 0, 0)),
        ],
        out_specs=pl.BlockSpec((1, BN, D), lambda r, nb: (r, nb, 0)),
        out_shape=jax.ShapeDtypeStruct((R, N, D), jnp.float32),
    )(x, basis)


def _sc_body(table, ridx3d, dst3d, slabs,
             acc, idxr, dstr, rows0, rows1, sem0, sem1):
    c = lax.axis_index("c")
    s = lax.axis_index("s")
    w = c * NS + s
    g0 = w * GPW

    def _zrow(i, carry):
        for j in range(D // L):
            rows0[i, pl.ds(j * L, L)] = jnp.zeros((L,), jnp.float32)
        return carry
    lax.fori_loop(0, CH, _zrow, 0)
    for t in range(ROWS_PER_TILE // CH):
        pltpu.sync_copy(rows0, acc.at[pl.ds(s * ROWS_PER_TILE + t * CH, CH)])

    pltpu.sync_copy(ridx3d.at[g0], idxr.at[0])
    pltpu.sync_copy(dst3d.at[g0], dstr.at[0])

    plsc.subcore_barrier()

    rowbufs = (rows0, rows1)
    sems = (sem0, sem1)

    def _gather(p, j, b):
        pltpu.async_copy(table.at[idxr.at[p, j]], rowbufs[b], sems[b])

    def _drain(p, j, b):
        pltpu.make_async_copy(table.at[idxr.at[p, j]],
                              rowbufs[b], sems[b]).wait()
        pltpu.sync_copy(rowbufs[b], acc.at[dstr.at[p, j]], add=True)

    def _group(g, p, stage_next, last):
        if stage_next:
            pltpu.sync_copy(ridx3d.at[g + 1], idxr.at[1 - p])
            pltpu.sync_copy(dst3d.at[g + 1], dstr.at[1 - p])
        for j in range(GRP):
            if not (last and j == GRP - 1):
                if j == GRP - 1:
                    _gather(1 - p, 0, (j + 1) % 2)
                else:
                    _gather(p, j + 1, (j + 1) % 2)
            _drain(p, j, j % 2)

    _gather(0, 0, 0)

    def _pair(m, carry):
        g = g0 + 2 * m
        _group(g, 0, True, False)
        _group(g + 1, 1, True, False)
        return carry
    lax.fori_loop(0, GPW // 2 - 1, _pair, 0)

    _group(g0 + GPW - 2, 0, True, False)
    _group(g0 + GPW - 1, 1, False, True)

    plsc.subcore_barrier()

    pltpu.sync_copy(acc.at[pl.ds(s * ROWS_PER_TILE, ROWS_PER_TILE)],
                    slabs.at[c, pl.ds(s * ROWS_PER_TILE, ROWS_PER_TILE)])


def _aggregate(table, ridx3d, dst3d):
    fn = pl.kernel(
        _sc_body,
        out_type=jax.ShapeDtypeStruct((NC, NP, D), jnp.float32),
        mesh=plsc.VectorSubcoreMesh(core_axis_name="c", subcore_axis_name="s"),
        scratch_types=[
            pltpu.VMEM_SHARED((NP, D), jnp.float32),
            pltpu.VMEM((2, GRP, CH), jnp.int32),
            pltpu.VMEM((2, GRP, CH), jnp.int32),
            pltpu.VMEM((CH, D), jnp.float32),
            pltpu.VMEM((CH, D), jnp.float32),
            pltpu.SemaphoreType.DMA,
            pltpu.SemaphoreType.DMA,
        ],
    )
    return fn(table, ridx3d, dst3d)


def _fin_body(s_ref, x_ref, root_ref, bias_ref, out_ref):
    out_ref[...] = (s_ref[0] + s_ref[1] + bias_ref[...]
                    + jnp.dot(x_ref[...], root_ref[...],
                              preferred_element_type=jnp.float32))


def _finalize(slabs, x, root, bias2d):
    return pl.pallas_call(
        _fin_body,
        grid=(N // BN,),
        in_specs=[
            pl.BlockSpec((NC, BN, D), lambda nb: (0, nb, 0)),
            pl.BlockSpec((BN, D), lambda nb: (nb, 0)),
            pl.BlockSpec((D, D), lambda nb: (0, 0)),
            pl.BlockSpec((1, D), lambda nb: (0, 0)),
        ],
        out_specs=pl.BlockSpec((BN, D), lambda nb: (nb, 0)),
        out_shape=jax.ShapeDtypeStruct((N, D), jnp.float32),
    )(slabs, x, root, bias2d)


def kernel(x, edge_index, edge_type, basis, root, bias):
    src = edge_index[0]
    dst = edge_index[1]
    ridx = edge_type * N + src
    pad = EP - E
    pad_src = jnp.arange(pad, dtype=jnp.int32) % (R * N)
    ridx3d = jnp.concatenate([ridx, pad_src]).reshape(NGRP, GRP, CH)
    pad_dst = N + jnp.arange(pad, dtype=jnp.int32) % (NP - N)
    dst3d = jnp.concatenate([dst, pad_dst]).reshape(NGRP, GRP, CH)

    table = _transform_table(x, basis).reshape(R * N, D)
    slabs = _aggregate(table, ridx3d, dst3d)
    return _finalize(slabs, x, root, bias.reshape(1, D))


# speedup vs baseline: 2.9452x; 1.0470x over previous
import jax
import jax.numpy as jnp
from jax import lax
from jax.experimental import pallas as pl
from jax.experimental.pallas import tpu as pltpu
from jax.experimental.pallas import tpu_sc as plsc

N, E, D, R = 10000, 320000, 128, 8

NC, NS, L = 2, 16, 16
NW = NC * NS
CH = 128
GRP = 8
CPW = 80
GPW = CPW // GRP
EP = NW * CPW * CH
NGRP = EP // (GRP * CH)
NP = 10240
ROWS_PER_TILE = NP // NS

BN = 2000


def _xw_body(x_ref, w_ref, y_ref):
    y_ref[0] = jnp.dot(x_ref[...], w_ref[0], preferred_element_type=jnp.float32)


def _transform_table(x, basis):
    return pl.pallas_call(
        _xw_body,
        grid=(N // BN, R),
        in_specs=[
            pl.BlockSpec((BN, D), lambda nb, r: (nb, 0)),
            pl.BlockSpec((1, D, D), lambda nb, r: (r, 0, 0)),
        ],
        out_specs=pl.BlockSpec((1, BN, D), lambda nb, r: (r, nb, 0)),
        out_shape=jax.ShapeDtypeStruct((R, N, D), jnp.float32),
    )(x, basis)


def _sc_body(table, ridx3d, dst3d, slabs,
             acc, idxr, dstr, rows0, rows1, sem0, sem1):
    c = lax.axis_index("c")
    s = lax.axis_index("s")
    w = c * NS + s
    g0 = w * GPW

    def _zrow(i, carry):
        for j in range(D // L):
            rows0[i, pl.ds(j * L, L)] = jnp.zeros((L,), jnp.float32)
        return carry
    lax.fori_loop(0, CH, _zrow, 0)
    for t in range(ROWS_PER_TILE // CH):
        pltpu.sync_copy(rows0, acc.at[pl.ds(s * ROWS_PER_TILE + t * CH, CH)])

    pltpu.sync_copy(ridx3d.at[g0], idxr.at[0])
    pltpu.sync_copy(dst3d.at[g0], dstr.at[0])

    plsc.subcore_barrier()

    rowbufs = (rows0, rows1)
    sems = (sem0, sem1)

    def _gather(p, j, b):
        pltpu.async_copy(table.at[idxr.at[p, j]], rowbufs[b], sems[b])

    def _drain(p, j, b):
        pltpu.make_async_copy(table.at[idxr.at[p, j]],
                              rowbufs[b], sems[b]).wait()
        pltpu.sync_copy(rowbufs[b], acc.at[dstr.at[p, j]], add=True)

    def _group(g, p, stage_next, last):
        if stage_next:
            pltpu.sync_copy(ridx3d.at[g + 1], idxr.at[1 - p])
            pltpu.sync_copy(dst3d.at[g + 1], dstr.at[1 - p])
        for j in range(GRP):
            if not (last and j == GRP - 1):
                if j == GRP - 1:
                    _gather(1 - p, 0, (j + 1) % 2)
                else:
                    _gather(p, j + 1, (j + 1) % 2)
            _drain(p, j, j % 2)

    _gather(0, 0, 0)

    def _pair(m, carry):
        g = g0 + 2 * m
        _group(g, 0, True, False)
        _group(g + 1, 1, True, False)
        return carry
    lax.fori_loop(0, GPW // 2 - 1, _pair, 0)

    _group(g0 + GPW - 2, 0, True, False)
    _group(g0 + GPW - 1, 1, False, True)

    plsc.subcore_barrier()

    pltpu.sync_copy(acc.at[pl.ds(s * ROWS_PER_TILE, ROWS_PER_TILE)],
                    slabs.at[c, pl.ds(s * ROWS_PER_TILE, ROWS_PER_TILE)])


def _aggregate(table, ridx3d, dst3d):
    fn = pl.kernel(
        _sc_body,
        out_type=jax.ShapeDtypeStruct((NC, NP, D), jnp.float32),
        mesh=plsc.VectorSubcoreMesh(core_axis_name="c", subcore_axis_name="s"),
        scratch_types=[
            pltpu.VMEM_SHARED((NP, D), jnp.float32),
            pltpu.VMEM((2, GRP, CH), jnp.int32),
            pltpu.VMEM((2, GRP, CH), jnp.int32),
            pltpu.VMEM((CH, D), jnp.float32),
            pltpu.VMEM((CH, D), jnp.float32),
            pltpu.SemaphoreType.DMA,
            pltpu.SemaphoreType.DMA,
        ],
    )
    return fn(table, ridx3d, dst3d)


def _fin_body(s_ref, x_ref, root_ref, bias_ref, out_ref):
    out_ref[...] = (s_ref[0] + s_ref[1] + bias_ref[...]
                    + jnp.dot(x_ref[...], root_ref[...],
                              preferred_element_type=jnp.float32))


def _finalize(slabs, x, root, bias2d):
    return pl.pallas_call(
        _fin_body,
        grid=(N // BN,),
        in_specs=[
            pl.BlockSpec((NC, BN, D), lambda nb: (0, nb, 0)),
            pl.BlockSpec((BN, D), lambda nb: (nb, 0)),
            pl.BlockSpec((D, D), lambda nb: (0, 0)),
            pl.BlockSpec((1, D), lambda nb: (0, 0)),
        ],
        out_specs=pl.BlockSpec((BN, D), lambda nb: (nb, 0)),
        out_shape=jax.ShapeDtypeStruct((N, D), jnp.float32),
    )(slabs, x, root, bias2d)


def kernel(x, edge_index, edge_type, basis, root, bias):
    src = edge_index[0]
    dst = edge_index[1]
    ridx = edge_type * N + src
    pad = EP - E
    pad_src = jnp.arange(pad, dtype=jnp.int32) % (R * N)
    ridx3d = jnp.concatenate([ridx, pad_src]).reshape(NGRP, GRP, CH)
    pad_dst = N + jnp.arange(pad, dtype=jnp.int32) % (NP - N)
    dst3d = jnp.concatenate([dst, pad_dst]).reshape(NGRP, GRP, CH)

    table = _transform_table(x, basis).reshape(R * N, D)
    slabs = _aggregate(table, ridx3d, dst3d)
    return _finalize(slabs, x, root, bias.reshape(1, D))
